# Initial kernel scaffold; baseline (speedup 1.0000x reference)
import jax
import jax.numpy as jnp
from jax.experimental import pallas as pl


def kernel(X, weight):
    raise NotImplementedError("write your pallas kernel here")


# speedup vs baseline: 1.5480x; 1.5480x over previous
import functools

import jax
import jax.numpy as jnp
from jax import lax
from jax.experimental import pallas as pl
from jax.experimental.pallas import tpu as pltpu
from jax.experimental.pallas import tpu_sc as plsc


@functools.cache
def _make_gather(V, D, B):
    info = plsc.get_sparse_core_info()
    NC, NS = info.num_cores, info.num_subcores
    NW = NC * NS
    assert B % NW == 0
    b_per_w = B // NW
    C = 1024
    assert b_per_w % C == 0
    n_chunks = b_per_w // C
    mesh = plsc.VectorSubcoreMesh(core_axis_name="c", subcore_axis_name="s")

    @functools.partial(
        pl.kernel,
        mesh=mesh,
        compiler_params=pltpu.CompilerParams(use_tc_tiling_on_sc=False),
        out_type=jax.ShapeDtypeStruct((B, D), jnp.float32),
        scratch_types=[
            pltpu.VMEM((C,), jnp.int32),
            pltpu.VMEM((C, D), jnp.float32),
            pltpu.SemaphoreType.DMA,
        ],
    )
    def gather_kernel(table_hbm, idx_hbm, out_hbm, idx_v, rows_v, sem):
        wid = lax.axis_index("s") * NC + lax.axis_index("c")
        base = wid * b_per_w

        def body(i, carry):
            off = base + i * C
            pltpu.sync_copy(idx_hbm.at[pl.ds(off, C)], idx_v)
            pltpu.async_copy(table_hbm.at[idx_v], rows_v, sem).wait()
            pltpu.sync_copy(rows_v, out_hbm.at[pl.ds(off, C)])
            return carry

        lax.fori_loop(0, n_chunks, body, 0)

    return gather_kernel


def kernel(X, weight):
    rows, cols = X.shape
    V, D = weight.shape
    B = rows * cols
    flat_idx = X.reshape(B).astype(jnp.int32)
    out = _make_gather(V, D, B)(weight, flat_idx)
    return out.reshape(rows, cols, D)


# speedup vs baseline: 1.5752x; 1.0176x over previous
import functools

import jax
import jax.numpy as jnp
from jax import lax
from jax.experimental import pallas as pl
from jax.experimental.pallas import tpu as pltpu
from jax.experimental.pallas import tpu_sc as plsc


@functools.cache
def _make_gather(V, D, B):
    info = plsc.get_sparse_core_info()
    NC, NS = info.num_cores, info.num_subcores
    NW = NC * NS
    assert B % NW == 0
    b_per_w = B // NW
    C = 1664
    assert b_per_w % C == 0
    n_chunks = b_per_w // C
    NB = 2
    mesh = plsc.VectorSubcoreMesh(core_axis_name="c", subcore_axis_name="s")

    @functools.partial(
        pl.kernel,
        mesh=mesh,
        compiler_params=pltpu.CompilerParams(use_tc_tiling_on_sc=False),
        out_type=jax.ShapeDtypeStruct((B, D), jnp.float32),
        scratch_types=[
            pltpu.VMEM((b_per_w,), jnp.int32),
            [pltpu.VMEM((C, D), jnp.float32) for _ in range(NB)],
            [pltpu.SemaphoreType.DMA for _ in range(NB)],
            [pltpu.SemaphoreType.DMA for _ in range(NB)],
        ],
    )
    def gather_kernel(table_hbm, idx_hbm, out_hbm, idx_all, rows, gsem, wsem):
        wid = lax.axis_index("s") * NC + lax.axis_index("c")
        base = wid * b_per_w

        pltpu.sync_copy(idx_hbm.at[pl.ds(base, b_per_w)], idx_all)

        def start_gather(i):
            return pltpu.async_copy(
                table_hbm.at[idx_all.at[pl.ds(i * C, C)]],
                rows[i % NB],
                gsem[i % NB],
            )

        def start_write(i):
            return pltpu.async_copy(
                rows[i % NB],
                out_hbm.at[pl.ds(base + i * C, C)],
                wsem[i % NB],
            )

        gcopy = [None] * n_chunks
        wcopy = [None] * n_chunks
        gcopy[0] = start_gather(0)
        for i in range(n_chunks):
            if i + 1 < n_chunks:
                if i + 1 >= NB:
                    wcopy[i + 1 - NB].wait()
                gcopy[i + 1] = start_gather(i + 1)
            gcopy[i].wait()
            wcopy[i] = start_write(i)
        for i in range(max(0, n_chunks - NB), n_chunks):
            wcopy[i].wait()

    return gather_kernel


def kernel(X, weight):
    rows, cols = X.shape
    V, D = weight.shape
    B = rows * cols
    flat_idx = X.reshape(B).astype(jnp.int32)
    out = _make_gather(V, D, B)(weight, flat_idx)
    return out.reshape(rows, cols, D)


# speedup vs baseline: 1.5808x; 1.0036x over previous
import functools

import jax
import jax.numpy as jnp
from jax import lax
from jax.experimental import pallas as pl
from jax.experimental.pallas import tpu as pltpu
from jax.experimental.pallas import tpu_sc as plsc


@functools.cache
def _make_gather(V, D, B):
    info = plsc.get_sparse_core_info()
    NC, NS = info.num_cores, info.num_subcores
    NW = NC * NS
    assert B % NW == 0
    b_per_w = B // NW
    C = 832
    assert b_per_w % C == 0
    n_chunks = b_per_w // C
    NB = 4
    mesh = plsc.VectorSubcoreMesh(core_axis_name="c", subcore_axis_name="s")

    @functools.partial(
        pl.kernel,
        mesh=mesh,
        compiler_params=pltpu.CompilerParams(use_tc_tiling_on_sc=False),
        out_type=jax.ShapeDtypeStruct((B, D), jnp.float32),
        scratch_types=[
            pltpu.VMEM((b_per_w,), jnp.int32),
            [pltpu.VMEM((C, D), jnp.float32) for _ in range(NB)],
            [pltpu.SemaphoreType.DMA for _ in range(NB)],
            [pltpu.SemaphoreType.DMA for _ in range(NB)],
        ],
    )
    def gather_kernel(table_hbm, idx_hbm, out_hbm, idx_all, rows, gsem, wsem):
        wid = lax.axis_index("s") * NC + lax.axis_index("c")
        base = wid * b_per_w

        pltpu.sync_copy(idx_hbm.at[pl.ds(base, b_per_w)], idx_all)

        def start_gather(i):
            return pltpu.async_copy(
                table_hbm.at[idx_all.at[pl.ds(i * C, C)]],
                rows[i % NB],
                gsem[i % NB],
            )

        def start_write(i):
            return pltpu.async_copy(
                rows[i % NB],
                out_hbm.at[pl.ds(base + i * C, C)],
                wsem[i % NB],
            )

        DEPTH = NB - 1
        gcopy = [None] * n_chunks
        wcopy = [None] * n_chunks
        for i in range(min(DEPTH, n_chunks)):
            gcopy[i] = start_gather(i)
        for i in range(n_chunks):
            gcopy[i].wait()
            wcopy[i] = start_write(i)
            j = i + DEPTH
            if j < n_chunks:
                if j >= NB:
                    wcopy[j - NB].wait()
                gcopy[j] = start_gather(j)
        for i in range(max(0, n_chunks - NB), n_chunks):
            wcopy[i].wait()

    return gather_kernel


def kernel(X, weight):
    rows, cols = X.shape
    V, D = weight.shape
    B = rows * cols
    flat_idx = X.reshape(B).astype(jnp.int32)
    out = _make_gather(V, D, B)(weight, flat_idx)
    return out.reshape(rows, cols, D)


# speedup vs baseline: 1.5993x; 1.0117x over previous
import functools

import jax
import jax.numpy as jnp
from jax import lax
from jax.experimental import pallas as pl
from jax.experimental.pallas import tpu as pltpu
from jax.experimental.pallas import tpu_sc as plsc


@functools.cache
def _make_gather(V, D, B):
    info = plsc.get_sparse_core_info()
    NC, NS = info.num_cores, info.num_subcores
    NW = NC * NS
    assert B % NW == 0
    b_per_w = B // NW
    C = 832
    assert b_per_w % C == 0
    n_chunks = b_per_w // C
    NB = 4
    mesh = plsc.VectorSubcoreMesh(core_axis_name="c", subcore_axis_name="s")

    @functools.partial(
        pl.kernel,
        mesh=mesh,
        compiler_params=pltpu.CompilerParams(use_tc_tiling_on_sc=False),
        out_type=jax.ShapeDtypeStruct((B, D), jnp.float32),
        scratch_types=[
            pltpu.VMEM((b_per_w,), jnp.int32),
            [pltpu.VMEM((C, D), jnp.float32) for _ in range(NB)],
            [pltpu.SemaphoreType.DMA for _ in range(NB)],
            [pltpu.SemaphoreType.DMA for _ in range(NB)],
        ],
    )
    def gather_kernel(table_hbm, idx_hbm, out_hbm, idx_all, rows, gsem, wsem):
        wid = lax.axis_index("s") * NC + lax.axis_index("c")
        base = wid * b_per_w

        pltpu.sync_copy(idx_hbm.at[pl.ds(base, b_per_w)], idx_all)

        def start_gather(i):
            return pltpu.async_copy(
                table_hbm.at[idx_all.at[pl.ds(i * C, C)]],
                rows[i % NB],
                gsem[i % NB],
            )

        def start_write(i):
            return pltpu.async_copy(
                rows[i % NB],
                out_hbm.at[pl.ds(base + i * C, C)],
                wsem[i % NB],
            )

        DEPTH = NB - 1
        gcopy = [None] * n_chunks
        wcopy = [None] * n_chunks
        for i in range(min(DEPTH, n_chunks)):
            gcopy[i] = start_gather(i)
        for i in range(n_chunks):
            gcopy[i].wait()
            wcopy[i] = start_write(i)
            j = i + DEPTH
            if j < n_chunks:
                if j >= NB:
                    wcopy[j - NB].wait()
                gcopy[j] = start_gather(j)
        for i in range(max(0, n_chunks - NB), n_chunks):
            wcopy[i].wait()

    return gather_kernel


def kernel(X, weight):
    rows, cols = X.shape
    V, D = weight.shape
    B = rows * cols
    flat_idx = X.reshape(B).astype(jnp.int32) * 4
    wview = jnp.pad(weight, ((0, 0), (0, 96))).reshape(V * 4, D)
    out = _make_gather(V * 4, D, B)(wview, flat_idx)
    return out.reshape(rows, cols, D)
